# baseline (speedup 1.0000x reference)
import functools

import jax
import jax.numpy as jnp
from jax import lax
from jax.experimental import pallas as pl
from jax.experimental.pallas import tpu as pltpu
from jax.experimental.pallas import tpu_sc as plsc

_VOCAB = 100000
_EMBED = 32
_BATCH = 1024
_VT = 2048
_NT = (_VOCAB + _VT - 1) // _VT


def _logits_tile(emb_ref, w_ref):
    return lax.dot_general(
        emb_ref[...], w_ref[...],
        dimension_numbers=(((1,), (1,)), ((), ())),
        preferred_element_type=jnp.float32,
        precision=lax.Precision.HIGHEST,
    )


def _stats_kernel(emb_ref, w_ref, m_ref, s_ref):
    i = pl.program_id(0)

    @pl.when(i == 0)
    def _init():
        m_ref[...] = jnp.full_like(m_ref, -jnp.inf)
        s_ref[...] = jnp.zeros_like(s_ref)

    logits = _logits_tile(emb_ref, w_ref)
    col = i * _VT + lax.broadcasted_iota(jnp.int32, (_BATCH, _VT), 1)
    logits = jnp.where(col < _VOCAB, logits, -jnp.inf)

    m_old = m_ref[...]
    m_new = jnp.maximum(m_old, jnp.max(logits, axis=1, keepdims=True))
    p_sum = jnp.sum(jnp.exp(logits - m_new), axis=1, keepdims=True)
    s_ref[...] = s_ref[...] * jnp.exp(m_old - m_new) + p_sum
    m_ref[...] = m_new


def _out_kernel(emb_ref, w_ref, m_ref, s_ref, o_ref):
    logits = _logits_tile(emb_ref, w_ref)
    o_ref[...] = jnp.exp(logits - m_ref[...]) * (1.0 / s_ref[...])


def _softmax_logits_tc(emb, out_w, interpret=False):
    emb_spec = pl.BlockSpec((_BATCH, _EMBED), lambda i: (0, 0))
    w_spec = pl.BlockSpec((_VT, _EMBED), lambda i: (i, 0))
    col_spec = pl.BlockSpec((_BATCH, 1), lambda i: (0, 0))
    params = pltpu.CompilerParams(dimension_semantics=("arbitrary",))

    m, s = pl.pallas_call(
        _stats_kernel,
        grid=(_NT,),
        in_specs=[emb_spec, w_spec],
        out_specs=[col_spec, col_spec],
        out_shape=[jax.ShapeDtypeStruct((_BATCH, 1), jnp.float32)] * 2,
        compiler_params=params,
        interpret=interpret,
    )(emb, out_w)

    return pl.pallas_call(
        _out_kernel,
        grid=(_NT,),
        in_specs=[emb_spec, w_spec, col_spec, col_spec],
        out_specs=pl.BlockSpec((_BATCH, _VT), lambda i: (0, i)),
        out_shape=jax.ShapeDtypeStruct((_BATCH, _VOCAB), jnp.float32),
        compiler_params=params,
        interpret=interpret,
    )(emb, out_w, m, s)


def _gather_rows_sc(table, idx):
    info = plsc.get_sparse_core_info()
    nc, ns = info.num_cores, info.num_subcores
    nw = nc * ns
    b_per_w = _BATCH // nw
    mesh = plsc.VectorSubcoreMesh(core_axis_name="c", subcore_axis_name="s")

    @functools.partial(
        pl.kernel,
        mesh=mesh,
        compiler_params=pltpu.CompilerParams(use_tc_tiling_on_sc=False),
        out_type=jax.ShapeDtypeStruct((_BATCH, _EMBED), jnp.float32),
        scratch_types=[
            pltpu.VMEM((b_per_w,), jnp.int32),
            pltpu.VMEM((b_per_w, _EMBED), jnp.float32),
            pltpu.SemaphoreType.DMA,
        ],
    )
    def gather_k(table_hbm, idx_hbm, out_hbm, idx_v, rows_v, sem):
        wid = lax.axis_index("s") * nc + lax.axis_index("c")
        base = wid * b_per_w
        pltpu.sync_copy(idx_hbm.at[pl.ds(base, b_per_w)], idx_v)
        pltpu.async_copy(table_hbm.at[idx_v], rows_v, sem).wait()
        pltpu.sync_copy(rows_v, out_hbm.at[pl.ds(base, b_per_w)])

    return gather_k(table, idx)


def kernel(input_labels, in_embed_weight, out_embed_weight):
    idx = input_labels.astype(jnp.int32)
    emb = _gather_rows_sc(in_embed_weight, idx)
    return _softmax_logits_tc(emb, out_embed_weight)


# speedup vs baseline: 1.6060x; 1.6060x over previous
import functools

import jax
import jax.numpy as jnp
from jax import lax
from jax.experimental import pallas as pl
from jax.experimental.pallas import tpu as pltpu
from jax.experimental.pallas import tpu_sc as plsc

_VOCAB = 100000
_EMBED = 32
_BATCH = 1024
_VT = 2048
_NT = (_VOCAB + _VT - 1) // _VT


def _logits_tile(emb_ref, w_ref):
    return lax.dot_general(
        emb_ref[...], w_ref[...],
        dimension_numbers=(((1,), (1,)), ((), ())),
        preferred_element_type=jnp.float32,
        precision=lax.Precision.DEFAULT,
    )


def _stats_kernel(emb_ref, w_ref, m_ref, s_ref):
    i = pl.program_id(0)

    @pl.when(i == 0)
    def _init():
        m_ref[...] = jnp.full_like(m_ref, -jnp.inf)
        s_ref[...] = jnp.zeros_like(s_ref)

    logits = _logits_tile(emb_ref, w_ref)
    col = i * _VT + lax.broadcasted_iota(jnp.int32, (_BATCH, _VT), 1)
    logits = jnp.where(col < _VOCAB, logits, -jnp.inf)

    m_old = m_ref[...]
    m_new = jnp.maximum(m_old, jnp.max(logits, axis=1, keepdims=True))
    p_sum = jnp.sum(jnp.exp(logits - m_new), axis=1, keepdims=True)
    s_ref[...] = s_ref[...] * jnp.exp(m_old - m_new) + p_sum
    m_ref[...] = m_new


def _out_kernel(emb_ref, w_ref, m_ref, s_ref, o_ref):
    logits = _logits_tile(emb_ref, w_ref)
    o_ref[...] = jnp.exp(logits - m_ref[...]) * (1.0 / s_ref[...])


def _softmax_logits_tc(emb, out_w, interpret=False):
    emb_spec = pl.BlockSpec((_BATCH, _EMBED), lambda i: (0, 0))
    w_spec = pl.BlockSpec((_VT, _EMBED), lambda i: (i, 0))
    col_spec = pl.BlockSpec((_BATCH, 1), lambda i: (0, 0))
    params = pltpu.CompilerParams(dimension_semantics=("arbitrary",))

    m, s = pl.pallas_call(
        _stats_kernel,
        grid=(_NT,),
        in_specs=[emb_spec, w_spec],
        out_specs=[col_spec, col_spec],
        out_shape=[jax.ShapeDtypeStruct((_BATCH, 1), jnp.float32)] * 2,
        compiler_params=params,
        interpret=interpret,
    )(emb, out_w)

    return pl.pallas_call(
        _out_kernel,
        grid=(_NT,),
        in_specs=[emb_spec, w_spec, col_spec, col_spec],
        out_specs=pl.BlockSpec((_BATCH, _VT), lambda i: (0, i)),
        out_shape=jax.ShapeDtypeStruct((_BATCH, _VOCAB), jnp.float32),
        compiler_params=params,
        interpret=interpret,
    )(emb, out_w, m, s)


def _gather_rows_sc(table, idx):
    info = plsc.get_sparse_core_info()
    nc, ns = info.num_cores, info.num_subcores
    nw = nc * ns
    b_per_w = _BATCH // nw
    mesh = plsc.VectorSubcoreMesh(core_axis_name="c", subcore_axis_name="s")

    @functools.partial(
        pl.kernel,
        mesh=mesh,
        compiler_params=pltpu.CompilerParams(use_tc_tiling_on_sc=False),
        out_type=jax.ShapeDtypeStruct((_BATCH, _EMBED), jnp.float32),
        scratch_types=[
            pltpu.VMEM((b_per_w,), jnp.int32),
            pltpu.VMEM((b_per_w, _EMBED), jnp.float32),
            pltpu.SemaphoreType.DMA,
        ],
    )
    def gather_k(table_hbm, idx_hbm, out_hbm, idx_v, rows_v, sem):
        wid = lax.axis_index("s") * nc + lax.axis_index("c")
        base = wid * b_per_w
        pltpu.sync_copy(idx_hbm.at[pl.ds(base, b_per_w)], idx_v)
        pltpu.async_copy(table_hbm.at[idx_v], rows_v, sem).wait()
        pltpu.sync_copy(rows_v, out_hbm.at[pl.ds(base, b_per_w)])

    return gather_k(table, idx)


def kernel(input_labels, in_embed_weight, out_embed_weight):
    idx = input_labels.astype(jnp.int32)
    emb = _gather_rows_sc(in_embed_weight, idx)
    return _softmax_logits_tc(emb, out_embed_weight)
